# baseline (speedup 1.0000x reference)
import functools

import jax
import jax.numpy as jnp
from jax import lax
from jax.experimental import pallas as pl
from jax.experimental.pallas import tpu as pltpu
from jax.experimental.pallas import tpu_sc as plsc

VOCAB = 1000000
EMBED_DIM = 64
BATCH = 4096
SEQ = 200
PAD = 1

N = BATCH * SEQ
NC, NS, LANES = 2, 16, 16
NW = NC * NS
PER_W = N // NW
CH = 128
NCH = PER_W // CH
NBUF = 2

_mesh = plsc.VectorSubcoreMesh(core_axis_name="c", subcore_axis_name="s")


@functools.partial(
    pl.kernel,
    mesh=_mesh,
    compiler_params=pltpu.CompilerParams(use_tc_tiling_on_sc=False),
    out_type=jax.ShapeDtypeStruct((N, EMBED_DIM), jnp.float32),
    scratch_types=[
        pltpu.VMEM((PER_W,), jnp.int32),
        pltpu.VMEM((PER_W,), jnp.int32),
        pltpu.VMEM((NBUF, CH, EMBED_DIM), jnp.float32),
        pltpu.SemaphoreType.DMA,
        pltpu.SemaphoreType.DMA,
    ],
)
def _emb_gather(x_hbm, m_hbm, table_hbm, out_hbm, idx_v, m_v, rows_v, sem0, sem1):
    sems = [sem0, sem1]
    wid = lax.axis_index("s") * NC + lax.axis_index("c")
    base = wid * PER_W

    pltpu.sync_copy(x_hbm.at[pl.ds(base, PER_W)], idx_v)
    pltpu.sync_copy(m_hbm.at[pl.ds(base, PER_W)], m_v)

    def _sel(i, _):
        off = i * LANES
        xs = idx_v[pl.ds(off, LANES)]
        ms = m_v[pl.ds(off, LANES)]
        idx_v[pl.ds(off, LANES)] = jnp.where(ms > 0, xs, PAD)
        return _

    lax.fori_loop(0, PER_W // LANES, _sel, 0)

    def _fire(j, s):
        return pltpu.async_copy(
            table_hbm.at[idx_v.at[pl.ds(j * CH, CH)]], rows_v.at[s], sems[s]
        )

    for s in range(NBUF):
        _fire(s, s)

    def _step(t, carry):
        for s in range(NBUF):
            j = t * NBUF + s
            pltpu.make_async_copy(
                table_hbm.at[idx_v.at[pl.ds(j * CH, CH)]], rows_v.at[s], sems[s]
            ).wait()
            pltpu.sync_copy(rows_v.at[s], out_hbm.at[pl.ds(base + j * CH, CH)])

            @pl.when(j + NBUF < NCH)
            def _fire_next():
                _fire(j + NBUF, s)

        return carry

    lax.fori_loop(0, NCH // NBUF, _step, 0)


def kernel(x, mask, table):
    out = _emb_gather(x.reshape(N), mask.reshape(N), table)
    return out.reshape(BATCH, SEQ, EMBED_DIM), mask


# speedup vs baseline: 6.8413x; 6.8413x over previous
import functools

import jax
import jax.numpy as jnp
from jax import lax
from jax.experimental import pallas as pl
from jax.experimental.pallas import tpu as pltpu
from jax.experimental.pallas import tpu_sc as plsc

VOCAB = 1000000
EMBED_DIM = 64
BATCH = 4096
SEQ = 200

N = BATCH * SEQ
NC, NS, LANES = 2, 16, 16
NW = NC * NS
PER_W = N // NW
CH = 128
NCH = PER_W // CH
BT = BATCH // CH
NRING = 4
GDEPTH = 2

_mesh = plsc.VectorSubcoreMesh(core_axis_name="c", subcore_axis_name="s")

_SPLAT_DNUMS = lax.GatherDimensionNumbers(
    offset_dims=(), collapsed_slice_dims=(0,), start_index_map=(0,))


def _splat(v, r):
    idx = jnp.full((16,), r, jnp.int32)
    return lax.gather(v, idx[:, None], _SPLAT_DNUMS, (1,),
                      mode=lax.GatherScatterMode.PROMISE_IN_BOUNDS)


@functools.partial(
    pl.kernel,
    mesh=_mesh,
    compiler_params=pltpu.CompilerParams(use_tc_tiling_on_sc=False),
    out_type=jax.ShapeDtypeStruct((SEQ, BATCH, EMBED_DIM), jnp.float32),
    scratch_types=[
        pltpu.VMEM((PER_W,), jnp.int32),
        pltpu.VMEM((PER_W,), jnp.int32),
        pltpu.VMEM((NRING, CH, EMBED_DIM), jnp.float32),
        pltpu.SemaphoreType.DMA((NRING,)),
        pltpu.SemaphoreType.DMA((NRING,)),
    ],
)
def _emb_gather(x_hbm, m_hbm, table_hbm, out_hbm, idx_v, m_v, rows_v,
                gsem, wsem):
    wid = lax.axis_index("s") * NC + lax.axis_index("c")
    base = wid * PER_W

    pltpu.sync_copy(x_hbm.at[pl.ds(base, PER_W)], idx_v)
    pltpu.sync_copy(m_hbm.at[pl.ds(base, PER_W)], m_v)

    def _gather(lc, s):
        return pltpu.make_async_copy(
            table_hbm.at[idx_v.at[pl.ds(lc * CH, CH)]], rows_v.at[s],
            gsem.at[s])

    def _write(lc, s):
        c = base // CH + lc
        cs = c // BT
        cbt = lax.rem(c, BT)
        return pltpu.make_async_copy(
            rows_v.at[s],
            out_hbm.at[cs, pl.ds(cbt * CH, CH), :], wsem.at[s])

    for j in range(GDEPTH):
        _gather(j, j % NRING).start()

    def _chunk(lc, s):
        _gather(lc, s).wait()

        def _mgroup(g, carry):
            mvec = jnp.where(m_v[pl.ds(lc * CH + g * LANES, LANES)] > 0,
                             jnp.float32(1.0), jnp.float32(0.0))
            for r in range(LANES):
                sp = _splat(mvec, r)
                row = g * LANES + r
                for kk in range(EMBED_DIM // LANES):
                    sl = pl.ds(kk * LANES, LANES)
                    rows_v[s, row, sl] = rows_v[s, row, sl] * sp
            return carry

        lax.fori_loop(0, CH // LANES, _mgroup, 0)

        _write(lc, s).start()

        @pl.when(lc + GDEPTH < NCH)
        def _prefetch():
            s2 = (lc + GDEPTH) % NRING

            @pl.when(lc >= NRING - GDEPTH)
            def _drain_prev_write():
                _write(lc - (NRING - GDEPTH), s2).wait()

            _gather(lc + GDEPTH, s2).start()

    def _outer(t, carry):
        for k in range(NRING):
            _chunk(t * NRING + k, k)
        return carry

    lax.fori_loop(0, NCH // NRING, _outer, 0)

    for j in range(NCH - NRING, NCH):
        _write(j, j % NRING).wait()


def kernel(x, mask, table):
    out = _emb_gather(x.T.reshape(N), mask.T.reshape(N), table)
    return out.transpose(1, 0, 2), mask
